# baseline (speedup 1.0000x reference)
import jax
import jax.numpy as jnp
from jax import lax
from jax.experimental import pallas as pl
from jax.experimental.pallas import tpu as pltpu
from jax.experimental.pallas import tpu_sc as plsc

N = 10000
E = 320000
D = 128

NC = 2
NS = 16
NW = NC * NS
CH = 64
NCH = 160
NP = 4
NCHP = NCH // NP
NR = 4
EPT = NCH * CH
E_PAD = NW * EPT
N_PAD = 10240
ROWS_PT = N_PAD // NS
ZCH = 64

_MESH = plsc.VectorSubcoreMesh(core_axis_name="c", subcore_axis_name="s")


def _sc_deg_body(eiw_hbm, deg_hbm, dst_all, zbuf, ones_v, deg_sp, sem):
    cid = lax.axis_index("c")
    sid = lax.axis_index("s")
    wid = cid * NS + sid
    for i in range(CH // 16):
        ones_v[pl.ds(i * 16, 16)] = jnp.full((16,), 1.0, jnp.float32)

    def zinit(i, carry):
        zbuf[pl.ds(i * 16, 16)] = jnp.zeros((16,), jnp.float32)
        return carry

    lax.fori_loop(0, ROWS_PT // 16, zinit, 0)
    pltpu.sync_copy(zbuf, deg_sp.at[pl.ds(sid * ROWS_PT, ROWS_PT)])
    pltpu.sync_copy(eiw_hbm.at[1, wid], dst_all)
    plsc.subcore_barrier()

    def fire(k, carry):
        pltpu.async_copy(ones_v, deg_sp.at[dst_all.at[k]], sem, add=True)
        return carry

    lax.fori_loop(0, NCH, fire, 0)

    def drain(k, carry):
        pltpu.make_async_copy(ones_v, deg_sp.at[dst_all.at[k]], sem).wait()
        return carry

    lax.fori_loop(0, NCH, drain, 0)
    plsc.subcore_barrier()
    pltpu.sync_copy(deg_sp.at[pl.ds(sid * ROWS_PT, ROWS_PT)],
                    deg_hbm.at[cid, pl.ds(sid * ROWS_PT, ROWS_PT)])


_sc_deg = pl.kernel(
    _sc_deg_body,
    out_type=jax.ShapeDtypeStruct((NC, N_PAD), jnp.float32),
    mesh=_MESH,
    scratch_types=[
        pltpu.VMEM((NCH, CH), jnp.int32),
        pltpu.VMEM((ROWS_PT,), jnp.float32),
        pltpu.VMEM((CH,), jnp.float32),
        pltpu.VMEM_SHARED((N_PAD,), jnp.float32),
        pltpu.SemaphoreType.DMA,
    ],
)


def _sc_agg_body(eiw_hbm, xs_hbm, z_hbm,
                 src_v, dst_v, rows, z_sp, s0, s1, s2, s3):
    sems = [s0, s1, s2, s3]
    cid = lax.axis_index("c")
    sid = lax.axis_index("s")
    wid = cid * NS + sid

    def zinit(i, carry):
        for j in range(D // 16):
            rows[0, i, pl.ds(j * 16, 16)] = jnp.zeros((16,), jnp.float32)
        return carry

    lax.fori_loop(0, ZCH, zinit, 0)
    for j in range(ROWS_PT // ZCH):
        pltpu.sync_copy(rows.at[0],
                        z_sp.at[pl.ds(sid * ROWS_PT + j * ZCH, ZCH)])
    plsc.subcore_barrier()

    def start_gather(c, b):
        pltpu.async_copy(xs_hbm.at[src_v.at[c]], rows.at[b], sems[b])

    def wait_gather(c, b):
        pltpu.make_async_copy(xs_hbm.at[src_v.at[c]], rows.at[b],
                              sems[b]).wait()

    def scatter(c, b):
        pltpu.sync_copy(rows.at[b], z_sp.at[dst_v.at[c]], add=True)

    def step(c, b, nxt):
        wait_gather(c, b)
        if nxt is not None:
            pb = (b + NR - 1) % NR
            start_gather(nxt, pb)
        scatter(c, b)

    for p in range(NP):
        pltpu.sync_copy(eiw_hbm.at[0, wid, pl.ds(p * NCHP, NCHP)], src_v)
        pltpu.sync_copy(eiw_hbm.at[1, wid, pl.ds(p * NCHP, NCHP)], dst_v)
        for c0 in range(NR - 1):
            start_gather(c0, c0)

        def group(g, carry):
            for b in range(NR):
                c = NR * g + b
                step(c, b, c + NR - 1)
            return carry

        lax.fori_loop(0, NCHP // NR - 1, group, 0)
        for b in range(NR):
            c = NCHP - NR + b
            step(c, b, c + NR - 1 if c + NR - 1 < NCHP else None)
    plsc.subcore_barrier()
    pltpu.sync_copy(z_sp.at[pl.ds(sid * ROWS_PT, ROWS_PT)],
                    z_hbm.at[cid, pl.ds(sid * ROWS_PT, ROWS_PT)])


_sc_agg = pl.kernel(
    _sc_agg_body,
    out_type=jax.ShapeDtypeStruct((NC, N_PAD, D), jnp.float32),
    mesh=_MESH,
    scratch_types=[
        pltpu.VMEM((NCHP, CH), jnp.int32),
        pltpu.VMEM((NCHP, CH), jnp.int32),
        pltpu.VMEM((NR, CH, D), jnp.float32),
        pltpu.VMEM_SHARED((N_PAD, D), jnp.float32),
        pltpu.SemaphoreType.DMA,
        pltpu.SemaphoreType.DMA,
        pltpu.SemaphoreType.DMA,
        pltpu.SemaphoreType.DMA,
    ],
)


def _tc_xs_body(x_ref, degp_ref, xs_ref):
    deg = degp_ref[0, :N] + degp_ref[1, :N] + 1.0
    dinv = lax.rsqrt(deg)
    xs_ref[...] = x_ref[...] * dinv[:, None]


_tc_xs = pl.pallas_call(
    _tc_xs_body,
    out_shape=jax.ShapeDtypeStruct((N, D), jnp.float32),
)


def _tc_out_body(x_ref, zp_ref, degp_ref, w1_ref, b1_ref, w2_ref, b2_ref,
                 o_ref):
    deg = degp_ref[0, :N] + degp_ref[1, :N] + 1.0
    dinv = lax.rsqrt(deg)[:, None]
    z = (zp_ref[0, :N] + zp_ref[1, :N]) * dinv + x_ref[...] * (dinv * dinv)
    h1 = jnp.dot(z, w1_ref[...], preferred_element_type=jnp.float32)
    h1 = jnp.maximum(h1 + b1_ref[...], 0.0)
    h2 = jnp.dot(z, w2_ref[...], preferred_element_type=jnp.float32)
    o_ref[...] = h1 + h2 + b2_ref[...]


_tc_out = pl.pallas_call(
    _tc_out_body,
    out_shape=jax.ShapeDtypeStruct((N, D), jnp.float32),
)


def kernel(x, edge_index, W1, b1, W2, b2):
    ei = edge_index.astype(jnp.int32)
    pad = E_PAD - E
    padblk = jnp.concatenate([jnp.zeros((1, pad), jnp.int32),
                              jnp.full((1, pad), N, jnp.int32)])
    eiw = jnp.concatenate([ei, padblk], axis=1).reshape(2, NW, NCH, CH)
    degp = _sc_deg(eiw)
    xs = _tc_xs(x, degp)
    zp = _sc_agg(eiw, xs)
    return _tc_out(x, zp, degp, W1, b1, W2, b2)
